# Initial kernel scaffold; baseline (speedup 1.0000x reference)
import jax
import jax.numpy as jnp
from jax.experimental import pallas as pl


def kernel(feat_0, feat_1, feat_dense, flow, space_mask, tau, use_sparsity, W_head, p_head, W_l0, p_l0, W_l1, p_l1, W_l2, p_l2, W_l3, p_l3, W_last, b_last, W_mask, b_mask):
    raise NotImplementedError("write your pallas kernel here")


# speedup vs baseline: 1.7714x; 1.7714x over previous
import functools
import jax
import jax.numpy as jnp
from jax import lax
from jax.experimental import pallas as pl
from jax.experimental.pallas import tpu as pltpu
from jax.experimental.pallas import tpu_sc as plsc

NF = 32
IN_NF = 100
CO = 36
H = 224
W = 224
HW = H * W
BH = 8
NBLK = H // BH
NCORN = 8
GROWS = NCORN * HW
NW = 32
PERW = GROWS // NW
CH = 784
NCHUNK = PERW // CH
TD = 128
CBH = 8


def _prep_body(flow_ref, idx_ref, w_ref):
    i = pl.program_id(0)
    rowf = (jax.lax.broadcasted_iota(jnp.int32, (BH, W), 0)
            + i * BH).astype(jnp.float32)
    colf = jax.lax.broadcasted_iota(jnp.int32, (BH, W), 1).astype(jnp.float32)
    for wi in range(2):
        vx = colf + flow_ref[:, :, 2 * wi]
        vy = rowf + flow_ref[:, :, 2 * wi + 1]
        x0 = jnp.floor(vx)
        y0 = jnp.floor(vy)
        ax = vx - x0
        ay = vy - y0
        for t, (cy, cx) in enumerate([(0, 0), (0, 1), (1, 0), (1, 1)]):
            xi = x0 + cx
            yi = y0 + cy
            valid = ((xi >= 0) & (xi <= W - 1) & (yi >= 0) & (yi <= H - 1))
            xc = jnp.clip(xi, 0, W - 1).astype(jnp.int32)
            yc = jnp.clip(yi, 0, H - 1).astype(jnp.int32)
            wx = ax if cx == 1 else (1.0 - ax)
            wy = ay if cy == 1 else (1.0 - ay)
            k = wi * 4 + t
            idx_ref[k] = yc * W + xc + wi * HW
            w_ref[k] = wx * wy * valid.astype(jnp.float32)


def _prep(flow_hwc):
    return pl.pallas_call(
        _prep_body,
        grid=(NBLK,),
        in_specs=[pl.BlockSpec((BH, W, 4), lambda i: (i, 0, 0))],
        out_specs=[
            pl.BlockSpec((NCORN, BH, W), lambda i: (0, i, 0)),
            pl.BlockSpec((NCORN, BH, W), lambda i: (0, i, 0)),
        ],
        out_shape=[
            jax.ShapeDtypeStruct((NCORN, H, W), jnp.int32),
            jax.ShapeDtypeStruct((NCORN, H, W), jnp.float32),
        ],
    )(flow_hwc)


def _gather_sc(table, idx_flat):
    mesh = plsc.VectorSubcoreMesh(core_axis_name="c", subcore_axis_name="s")

    @functools.partial(
        pl.kernel,
        mesh=mesh,
        out_type=jax.ShapeDtypeStruct((GROWS, TD), jnp.float32),
        scratch_types=[
            pltpu.VMEM((CH,), jnp.int32),
            pltpu.VMEM((CH, TD), jnp.float32),
            pltpu.SemaphoreType.DMA,
        ],
    )
    def k(table_hbm, idx_hbm, out_hbm, idx_v, rows_v, sem):
        wid = lax.axis_index("s") * 2 + lax.axis_index("c")
        base = wid * PERW

        def body(c, carry):
            off = base + c * CH
            pltpu.sync_copy(idx_hbm.at[pl.ds(off, CH)], idx_v)
            pltpu.async_copy(table_hbm.at[idx_v], rows_v, sem).wait()
            pltpu.sync_copy(rows_v, out_hbm.at[pl.ds(off, CH)])
            return carry

        lax.fori_loop(0, NCHUNK, body, 0)

    return k(table, idx_flat)


def _combine_body(g_ref, w_ref, fd_ref, fl_ref, x_ref):
    parts = []
    for wi in range(2):
        acc = jnp.zeros((CBH, W, NF), jnp.float32)
        for t in range(4):
            k = wi * 4 + t
            acc = acc + g_ref[k][:, :, :NF] * w_ref[k][:, :, None]
        parts.append(acc)
    parts.append(fd_ref[...])
    parts.append(fl_ref[...])
    x_ref[...] = jnp.concatenate(parts, axis=-1)


def _combine(g4, w8, fd, fl):
    return pl.pallas_call(
        _combine_body,
        grid=(H // CBH,),
        in_specs=[
            pl.BlockSpec((NCORN, CBH, W, TD), lambda i: (0, i, 0, 0)),
            pl.BlockSpec((NCORN, CBH, W), lambda i: (0, i, 0)),
            pl.BlockSpec((CBH, W, NF), lambda i: (i, 0, 0)),
            pl.BlockSpec((CBH, W, 4), lambda i: (i, 0, 0)),
        ],
        out_specs=pl.BlockSpec((CBH, W, IN_NF), lambda i: (i, 0, 0)),
        out_shape=jax.ShapeDtypeStruct((H, W, IN_NF), jnp.float32),
    )(g4, w8, fd, fl)


def _conv9(xm1_ref, x0_ref, xp1_ref, wt_ref, cin, cout):
    i = pl.program_id(0)
    mt = jnp.where(i > 0, 1.0, 0.0)
    mb = jnp.where(i < NBLK - 1, 1.0, 0.0)
    top = xm1_ref[BH - 1:BH] * mt
    bot = xp1_ref[0:1] * mb
    xv = jnp.concatenate([top, x0_ref[...], bot], axis=0)
    zc = jnp.zeros((BH + 2, 1, cin), jnp.float32)
    xp = jnp.concatenate([zc, xv, zc], axis=1)
    if cin < 128:
        xp = jnp.concatenate(
            [xp, jnp.zeros((BH + 2, W + 2, 128 - cin), jnp.float32)], axis=-1)
    cols = jnp.concatenate([xp[ky:ky + BH, kx:kx + W, :].reshape(BH * W, 128)
                            for ky in range(3) for kx in range(3)], axis=1)
    z = jnp.dot(cols, wt_ref[...], preferred_element_type=jnp.float32)
    return z.reshape(BH, W, cout)


def _conv_head_body(xm1, x0, xp1, wt, p, sm, wl, bl, y_ref, acc_ref):
    z = _conv9(xm1, x0, xp1, wt, IN_NF, IN_NF)
    y = jnp.maximum(z, 0) + p[0, 0] * jnp.minimum(z, 0)
    y = y * sm[...]
    y_ref[...] = y
    part = jnp.dot(y.reshape(BH * W, IN_NF), wl[0],
                   preferred_element_type=jnp.float32)
    acc_ref[...] = part.reshape(BH, W, CO) + bl[0, 0]


def _conv_layer_body(xm1, x0, xp1, wt, p, sm, wl, accin, y_ref, acc_ref):
    z = _conv9(xm1, x0, xp1, wt, IN_NF, IN_NF)
    z = z * sm[...]
    y = jnp.maximum(z, 0) + p[0, 0] * jnp.minimum(z, 0)
    y_ref[...] = y
    part = jnp.dot(y.reshape(BH * W, IN_NF), wl[0],
                   preferred_element_type=jnp.float32)
    acc_ref[...] = accin[...] + part.reshape(BH, W, CO)


def _x3specs(c):
    return [
        pl.BlockSpec((BH, W, c), lambda i: (jnp.maximum(i - 1, 0), 0, 0)),
        pl.BlockSpec((BH, W, c), lambda i: (i, 0, 0)),
        pl.BlockSpec((BH, W, c),
                     lambda i: (jnp.minimum(i + 1, NBLK - 1), 0, 0)),
    ]


_FULL3 = lambda s: pl.BlockSpec(s, lambda i: (0, 0, 0))
_FULL2 = lambda s: pl.BlockSpec(s, lambda i: (0, 0))


def _conv_head(x, wt, p, sm, wl0, bl):
    return pl.pallas_call(
        _conv_head_body,
        grid=(NBLK,),
        in_specs=_x3specs(IN_NF) + [
            _FULL2((9 * 128, IN_NF)),
            _FULL3((1, 1, IN_NF)),
            pl.BlockSpec((BH, W, 1), lambda i: (i, 0, 0)),
            _FULL3((1, IN_NF, CO)),
            _FULL3((1, 1, CO)),
        ],
        out_specs=[
            pl.BlockSpec((BH, W, IN_NF), lambda i: (i, 0, 0)),
            pl.BlockSpec((BH, W, CO), lambda i: (i, 0, 0)),
        ],
        out_shape=[
            jax.ShapeDtypeStruct((H, W, IN_NF), jnp.float32),
            jax.ShapeDtypeStruct((H, W, CO), jnp.float32),
        ],
    )(x, x, x, wt, p, sm, wl0, bl)


def _conv_layer(x, wt, p, sm, wlk, accin):
    return pl.pallas_call(
        _conv_layer_body,
        grid=(NBLK,),
        in_specs=_x3specs(IN_NF) + [
            _FULL2((9 * 128, IN_NF)),
            _FULL3((1, 1, IN_NF)),
            pl.BlockSpec((BH, W, 1), lambda i: (i, 0, 0)),
            _FULL3((1, IN_NF, CO)),
            pl.BlockSpec((BH, W, CO), lambda i: (i, 0, 0)),
        ],
        out_specs=[
            pl.BlockSpec((BH, W, IN_NF), lambda i: (i, 0, 0)),
            pl.BlockSpec((BH, W, CO), lambda i: (i, 0, 0)),
        ],
        out_shape=[
            jax.ShapeDtypeStruct((H, W, IN_NF), jnp.float32),
            jax.ShapeDtypeStruct((H, W, CO), jnp.float32),
        ],
    )(x, x, x, wt, p, sm, wlk, accin)


def _conv_smn_body(xm1, x0, xp1, wt, b, smn_ref):
    z = _conv9(xm1, x0, xp1, wt, NF, 2)
    smn_ref[...] = z + b[0, 0]


def _conv_smn(f32ch, wt, b):
    return pl.pallas_call(
        _conv_smn_body,
        grid=(NBLK,),
        in_specs=_x3specs(NF) + [_FULL2((9 * 128, 2)), _FULL3((1, 1, 2))],
        out_specs=pl.BlockSpec((BH, W, 2), lambda i: (i, 0, 0)),
        out_shape=jax.ShapeDtypeStruct((H, W, 2), jnp.float32),
    )(f32ch, f32ch, f32ch, wt, b)


PBH = 8
NPB = H // PBH


def _updown(xm1, x0, xp1, i):
    mt = jnp.where(i > 0, 1.0, 0.0)
    mb = jnp.where(i < NPB - 1, 1.0, 0.0)
    top = xm1[PBH - 1:PBH] * mt + x0[0:1] * (1.0 - mt)
    bot = xp1[0:1] * mb + x0[PBH - 1:PBH] * (1.0 - mb)
    xv = jnp.concatenate([top, x0[...], bot], axis=0)
    up = 0.25 * xv[0:PBH] + 0.75 * xv[1:PBH + 1]
    dn = 0.75 * xv[1:PBH + 1] + 0.25 * xv[2:PBH + 2]
    return up, dn


def _lr(u):
    xl = jnp.concatenate([u[:, 0:1], u[:, 0:W - 1]], axis=1)
    xr = jnp.concatenate([u[:, 1:W], u[:, W - 1:W]], axis=1)
    return 0.25 * xl + 0.75 * u, 0.75 * u + 0.25 * xr


def _post_body(fm1, f0, fp1, sm1, s0, sp1, smref,
               p00, p01, p10, p11, m00, m01, m10, m11):
    i = pl.program_id(0)
    fu, fd = _updown(fm1, f0, fp1, i)
    su, sd = _updown(sm1, s0, sp1, i)
    sm = smref[...]
    for (fv, sv, pe, po, me, mo) in (
            (fu, su, p00, p01, m00, m01),
            (fd, sd, p10, p11, m10, m11)):
        fe, fo = _lr(fv)
        se, so = _lr(sv)
        pe[...] = fe
        po[...] = fo
        me[...] = (se[:, :, 0:1] > se[:, :, 1:2]).astype(jnp.float32) * sm
        mo[...] = (so[:, :, 0:1] > so[:, :, 1:2]).astype(jnp.float32) * sm


def _p3specs(c):
    return [
        pl.BlockSpec((PBH, W, c), lambda i: (jnp.maximum(i - 1, 0), 0, 0)),
        pl.BlockSpec((PBH, W, c), lambda i: (i, 0, 0)),
        pl.BlockSpec((PBH, W, c),
                     lambda i: (jnp.minimum(i + 1, NPB - 1), 0, 0)),
    ]


def _post(final, smn, sm):
    blk36 = pl.BlockSpec((PBH, W, CO), lambda i: (i, 0, 0))
    blk1 = pl.BlockSpec((PBH, W, 1), lambda i: (i, 0, 0))
    return pl.pallas_call(
        _post_body,
        grid=(NPB,),
        in_specs=_p3specs(CO) + _p3specs(2) + [blk1],
        out_specs=[blk36] * 4 + [blk1] * 4,
        out_shape=[jax.ShapeDtypeStruct((H, W, CO), jnp.float32)] * 4
        + [jax.ShapeDtypeStruct((H, W, 1), jnp.float32)] * 4,
    )(final, final, final, smn, smn, smn, sm)


def _interleave(p00, p01, p10, p11):
    c = p00.shape[-1]
    x = jnp.stack([p00, p01, p10, p11], axis=0).reshape(2, 2, H, W, c)
    return x.transpose(2, 0, 3, 1, 4).reshape(2 * H, 2 * W, c)


def _tap(wconv):
    t = wconv.transpose(2, 3, 1, 0)
    t = jnp.pad(t, ((0, 0), (0, 0), (0, 128 - t.shape[2]), (0, 0)))
    return t.reshape(9 * 128, wconv.shape[0])


def kernel(feat_0, feat_1, feat_dense, flow, space_mask, tau, use_sparsity,
           W_head, p_head, W_l0, p_l0, W_l1, p_l1, W_l2, p_l2, W_l3, p_l3,
           W_last, b_last, W_mask, b_mask):
    table = jnp.pad(jnp.concatenate([
        feat_0[0].transpose(1, 2, 0).reshape(HW, NF),
        feat_1[0].transpose(1, 2, 0).reshape(HW, NF)], axis=0),
        ((0, 0), (0, TD - NF)))
    fd = feat_dense[0].transpose(1, 2, 0)
    fl = flow[0].transpose(1, 2, 0)
    sm = space_mask[0, 0][:, :, None]
    wt_head = _tap(W_head)
    wts = [_tap(W_l0), _tap(W_l1), _tap(W_l2), _tap(W_l3)]
    pps = [p_l0, p_l1, p_l2, p_l3]
    wlast = W_last.reshape(CO, 5 * IN_NF).T.reshape(5, 1, IN_NF, CO)
    blast = b_last.reshape(1, 1, CO)
    wt_mask = _tap(W_mask)
    bmask = b_mask.reshape(1, 1, 2)

    idx8, w8 = _prep(fl)
    g = _gather_sc(table, idx8.reshape(GROWS))
    g4 = g.reshape(NCORN, H, W, TD)

    x = _combine(g4, w8, fd, fl)
    y, acc = _conv_head(x, wt_head, p_head.reshape(1, 1, IN_NF), sm,
                        wlast[0], blast)
    for k in range(4):
        y, acc = _conv_layer(y, wts[k], pps[k].reshape(1, 1, IN_NF), sm,
                             wlast[k + 1], acc)

    f32ch = acc[:, :, 4:CO]
    smn = _conv_smn(f32ch, wt_mask, bmask)
    p00, p01, p10, p11, m00, m01, m10, m11 = _post(acc, smn, sm)

    xup = _interleave(p00, p01, p10, p11)
    mup = _interleave(m00, m01, m10, m11)
    xup = xup.transpose(2, 0, 1)[None]
    flow_out = xup[:, :4]
    feat_t = xup[:, 4:]
    smn_final = mup.transpose(2, 0, 1)[None]
    return (flow_out, feat_t, smn_final)
